# baseline (speedup 1.0000x reference)
import functools

import jax
import jax.numpy as jnp
from jax import lax
from jax.experimental import pallas as pl
from jax.experimental.pallas import tpu as pltpu
from jax.experimental.pallas import tpu_sc as plsc

HIDDEN = 768
BATCH = 4096
SEQ = 77
SEQ_PAD = 80
NW = 32
SPW = BATCH // NW
LANES = 16
VPR = HIDDEN // LANES

SLABS = ((0, 32), (32, 32), (64, 16))
NSEC = len(SLABS)


def _embed_kernel(ids_hbm, tab_hbm, pos_hbm, out_hbm, idx_v, pos_v, *rest):
    bufs = rest[:NSEC]
    gsem = rest[NSEC:2 * NSEC]
    osem = rest[2 * NSEC:3 * NSEC]
    wid = lax.axis_index("s") * 2 + lax.axis_index("c")
    pltpu.sync_copy(pos_hbm, pos_v)
    pltpu.sync_copy(ids_hbm.at[pl.ds(wid * SPW * SEQ_PAD, SPW * SEQ_PAD)],
                    idx_v)

    def start_gather(s, i):
        row0, nrows = SLABS[i]
        pltpu.async_copy(
            tab_hbm.at[idx_v.at[pl.ds(s * SEQ_PAD + row0, nrows)]],
            bufs[i], gsem[i])

    def wait_gather(i):
        row0, nrows = SLABS[i]
        pltpu.make_async_copy(tab_hbm.at[pl.ds(0, nrows)], bufs[i],
                              gsem[i]).wait()

    def drain_out(j):
        row0, nrows = SLABS[j]
        pltpu.make_async_copy(bufs[j], out_hbm.at[0, pl.ds(row0, nrows)],
                              osem[j]).wait()

    def add_rows(buf, nrows, row0):
        def row(r, carry):
            pb = (row0 + r) * HIDDEN
            for g in range(0, VPR, 16):
                vals = [pos_v[pl.ds(pb + (g + c) * LANES, LANES)]
                        for c in range(16)]
                for c in range(16):
                    plsc.addupdate(buf.at[r, pl.ds((g + c) * LANES, LANES)],
                                   vals[c])
            return carry
        lax.fori_loop(0, nrows, row, 0)

    def body(s, carry):
        gb = wid * SPW + s
        for i in range(NSEC):
            row0, nrows = SLABS[i]
            wait_gather(i)
            add_rows(bufs[i], min(nrows, SEQ - row0), row0)
            pltpu.async_copy(bufs[i], out_hbm.at[gb, pl.ds(row0, nrows)],
                             osem[i])
            if i <= NSEC - 3:
                @pl.when(s > 0)
                def _drain():
                    drain_out(i + 2)
                start_gather(s, i + 2)
            else:
                @pl.when(s < SPW - 1)
                def _refill():
                    drain_out(i - (NSEC - 2))
                    start_gather(s + 1, i - (NSEC - 2))
        return carry

    start_gather(0, 0)
    start_gather(0, 1)
    lax.fori_loop(0, SPW, body, 0)
    for i in range(NSEC):
        drain_out(i)


def kernel(input_ids, token_embedding, position_embedding):
    ids = input_ids.astype(jnp.int32)
    ids_pad = jnp.pad(ids, ((0, 0), (0, SEQ_PAD - SEQ))).reshape(-1)
    pos_flat = position_embedding.reshape(-1)

    mesh = plsc.VectorSubcoreMesh(core_axis_name="c", subcore_axis_name="s")
    run = functools.partial(
        pl.kernel,
        mesh=mesh,
        out_type=jax.ShapeDtypeStruct((BATCH, SEQ_PAD, HIDDEN), jnp.float32),
        scratch_types=[
            pltpu.VMEM((SPW * SEQ_PAD,), jnp.int32),
            pltpu.VMEM((SEQ * HIDDEN,), jnp.float32),
        ] + [pltpu.VMEM((n, HIDDEN), jnp.float32) for _, n in SLABS]
          + [pltpu.SemaphoreType.DMA] * (2 * NSEC),
    )(_embed_kernel)
    return run(ids_pad, token_embedding, pos_flat)[:, :SEQ, :]
